# baseline (speedup 1.0000x reference)
import functools

import jax
import jax.numpy as jnp
from jax import lax
from jax.experimental import pallas as pl
from jax.experimental.pallas import tpu as pltpu
from jax.experimental.pallas import tpu_sc as plsc

C = 1000
CP = 1024
TAIL0 = 896
TAILW = C - TAIL0
N = 51200
NC, NS, L = 2, 16, 16
NW = NC * NS
PER_W = N // NW
GW = 32
LZW = 64


def _vector_mesh():
    return plsc.VectorSubcoreMesh(core_axis_name="c", subcore_axis_name="s")


def _prep_body(t_ref, lz_ref, pad_ref):
    x = t_ref[...]
    m = jnp.max(x, axis=1)
    s = jnp.sum(jnp.exp(x - m[:, None]), axis=1)
    lz_ref[...] = m + jnp.log(s)
    pad_ref[...] = jnp.concatenate(
        [x, jnp.zeros((C, CP - C), jnp.float32)], axis=1)


def _prep(table):
    return pl.pallas_call(
        _prep_body,
        out_shape=(
            jax.ShapeDtypeStruct((C,), jnp.float32),
            jax.ShapeDtypeStruct((C, CP), jnp.float32),
        ),
    )(table)


def _gather_and_parts(table_pad, rowlz, idx_flat, tgt_flat):
    n_chunks = PER_W // GW

    @functools.partial(
        pl.kernel,
        out_type=(
            jax.ShapeDtypeStruct((N, C), jnp.float32),
            jax.ShapeDtypeStruct((NW, L), jnp.float32),
        ),
        mesh=_vector_mesh(),
        compiler_params=pltpu.CompilerParams(needs_layout_passes=False),
        scratch_types=[
            pltpu.VMEM((PER_W,), jnp.int32),
            pltpu.VMEM((PER_W,), jnp.int32),
            pltpu.VMEM((PER_W,), jnp.float32),
            pltpu.VMEM((L,), jnp.float32),
            pltpu.VMEM((GW, C), jnp.float32),
            pltpu.VMEM((GW, C), jnp.float32),
            pltpu.VMEM((GW, 128), jnp.float32),
            pltpu.VMEM((GW, 128), jnp.float32),
            pltpu.SemaphoreType.DMA,
            pltpu.SemaphoreType.DMA,
            pltpu.SemaphoreType.DMA,
            pltpu.SemaphoreType.DMA,
            pltpu.SemaphoreType.DMA,
        ],
    )
    def k(table_hbm, lz_hbm, idx_hbm, tgt_hbm, out_hbm, parts_hbm,
          idx_v, tgt_v, lz_v, acc_v, rows0, rows1, tail0, tail1,
          g0, g1, s0, s1, lzsem):
        wid = lax.axis_index("s") * NC + lax.axis_index("c")
        base = wid * PER_W
        pltpu.sync_copy(idx_hbm.at[pl.ds(base, PER_W)], idx_v)
        pltpu.sync_copy(tgt_hbm.at[pl.ds(base, PER_W)], tgt_v)

        @pl.loop(0, PER_W, step=LZW)
        def _(j):
            sl = pl.ds(j, LZW)
            pltpu.make_async_copy(
                lz_hbm.at[idx_v.at[sl]], lz_v.at[sl], lzsem).start()

        rows = (rows0, rows1)
        tails = (tail0, tail1)
        gsem = (g0, g1)
        ssem = (s0, s1)

        def _gather_copies(c, b):
            isl = idx_v.at[pl.ds(c * GW, GW)]
            cps = []
            for t in range(7):
                cs = pl.ds(t * 128, 128)
                cps.append(pltpu.make_async_copy(
                    table_hbm.at[:, cs].at[isl], rows[b].at[:, cs], gsem[b]))
            cps.append(pltpu.make_async_copy(
                table_hbm.at[:, pl.ds(TAIL0, 128)].at[isl], tails[b], gsem[b]))
            return cps

        def _write_copies(c, b):
            dst_rows = pl.ds(base + c * GW, GW)
            cps = []
            for t in range(7):
                cs = pl.ds(t * 128, 128)
                cps.append(pltpu.make_async_copy(
                    rows[b].at[:, cs], out_hbm.at[dst_rows, cs], ssem[b]))
            ct = pl.ds(TAIL0, TAILW)
            cps.append(pltpu.make_async_copy(
                rows[b].at[:, ct], out_hbm.at[dst_rows, ct], ssem[b]))
            return cps

        def gather_start(c, b):
            for cp in _gather_copies(c, b):
                cp.start()

        def gather_wait(c, b):
            for cp in _gather_copies(c, b):
                cp.wait()

        def write_start(c, b):
            for cp in _write_copies(c, b):
                cp.start()

        def write_wait(c, b):
            for cp in _write_copies(c, b):
                cp.wait()

        gather_start(0, 0)
        gather_start(1, 1)
        acc_v[...] = jnp.zeros((L,), jnp.float32)

        @pl.loop(0, n_chunks, step=2)
        def _(c0):
            for b in range(2):
                c = c0 + b
                gather_wait(c, b)

                @pl.loop(0, GW)
                def _(r):
                    for kk in (0, 16, 32, 48, 64, 80, TAILW - 16):
                        rows[b][r, pl.ds(TAIL0 + kk, 16)] = (
                            tails[b][r, pl.ds(kk, 16)])

                write_start(c, b)
                for j in range(0, GW, L):
                    rowi = jnp.arange(L, dtype=jnp.int32) + j
                    colt = tgt_v[pl.ds(c * GW + j, L)]
                    vals = plsc.load_gather(rows[b], [rowi, colt])
                    acc_v[...] = acc_v[...] - vals
            for b in range(2):
                nxt = c0 + 2 + b

                @pl.when(nxt < n_chunks)
                def _():
                    write_wait(c0 + b, b)
                    gather_start(nxt, b)

        @pl.loop(0, PER_W, step=LZW)
        def _(j):
            sl = pl.ds(j, LZW)
            pltpu.make_async_copy(
                lz_hbm.at[idx_v.at[sl]], lz_v.at[sl], lzsem).wait()

        @pl.loop(0, PER_W, step=L)
        def _(j):
            acc_v[...] = acc_v[...] + lz_v[pl.ds(j, L)]

        pltpu.sync_copy(acc_v, parts_hbm.at[wid])
        write_wait(n_chunks - 2, 0)
        write_wait(n_chunks - 1, 1)

    return k(table_pad, rowlz, idx_flat, tgt_flat)


def _tr_body(x_ref, o_ref):
    o_ref[...] = x_ref[...].T


def _transpose(h):
    n = h.shape[0]
    return pl.pallas_call(
        _tr_body,
        grid=(n // 128,),
        in_specs=[pl.BlockSpec((128, C), lambda i: (i, 0))],
        out_specs=pl.BlockSpec((C, 128), lambda i: (0, i)),
        out_shape=jax.ShapeDtypeStruct((C, n), jnp.float32),
    )(h)


def _reduce_body(p_ref, o_ref):
    o_ref[...] = (jnp.sum(p_ref[...]) / N).reshape(1, 1)


def _reduce_loss(parts):
    return pl.pallas_call(
        _reduce_body,
        out_shape=jax.ShapeDtypeStruct((1, 1), jnp.float32),
    )(parts)


def kernel(idx, targets, table):
    idx_flat = idx.reshape(-1).astype(jnp.int32)
    tgt_flat = targets.reshape(-1).astype(jnp.int32)
    rowlz, table_pad = _prep(table)
    logits_rm, parts = _gather_and_parts(table_pad, rowlz, idx_flat, tgt_flat)
    logits2 = _transpose(logits_rm).T
    loss = _reduce_loss(parts)
    return (logits2, loss[0, 0])


# speedup vs baseline: 1.5204x; 1.5204x over previous
import functools

import jax
import jax.numpy as jnp
from jax import lax
from jax.experimental import pallas as pl
from jax.experimental.pallas import tpu as pltpu
from jax.experimental.pallas import tpu_sc as plsc

C = 1000
CP = 1024
TAIL0 = 896
TAILW = C - TAIL0
N = 51200
NC, NS, L = 2, 16, 16
NW = NC * NS
PER_W = N // NW
GW = 32
LZW = 64


def _vector_mesh():
    return plsc.VectorSubcoreMesh(core_axis_name="c", subcore_axis_name="s")


def _prep_body(t_ref, lz_ref, pad_ref):
    x = t_ref[...]
    m = jnp.max(x, axis=1)
    s = jnp.sum(jnp.exp(x - m[:, None]), axis=1)
    lz_ref[...] = m + jnp.log(s)
    pad_ref[...] = jnp.concatenate(
        [x, jnp.zeros((C, CP - C), jnp.float32)], axis=1)


def _prep(table):
    return pl.pallas_call(
        _prep_body,
        out_shape=(
            jax.ShapeDtypeStruct((C,), jnp.float32),
            jax.ShapeDtypeStruct((C, CP), jnp.float32),
        ),
    )(table)


def _gather_and_parts(table_pad, rowlz, idx_flat, tgt_flat):
    n_chunks = PER_W // GW

    @functools.partial(
        pl.kernel,
        out_type=(
            jax.ShapeDtypeStruct((N, C), jnp.float32),
            jax.ShapeDtypeStruct((NW, L), jnp.float32),
        ),
        mesh=_vector_mesh(),
        compiler_params=pltpu.CompilerParams(needs_layout_passes=False),
        scratch_types=[
            pltpu.VMEM((PER_W,), jnp.int32),
            pltpu.VMEM((PER_W,), jnp.int32),
            pltpu.VMEM((PER_W,), jnp.float32),
            pltpu.VMEM((L,), jnp.float32),
            pltpu.VMEM((GW, C), jnp.float32),
            pltpu.VMEM((GW, C), jnp.float32),
            pltpu.VMEM((GW, 128), jnp.float32),
            pltpu.VMEM((GW, 128), jnp.float32),
            pltpu.SemaphoreType.DMA,
            pltpu.SemaphoreType.DMA,
            pltpu.SemaphoreType.DMA,
            pltpu.SemaphoreType.DMA,
            pltpu.SemaphoreType.DMA,
        ],
    )
    def k(table_hbm, lz_hbm, idx_hbm, tgt_hbm, out_hbm, parts_hbm,
          idx_v, tgt_v, lz_v, acc_v, rows0, rows1, tail0, tail1,
          g0, g1, s0, s1, lzsem):
        wid = lax.axis_index("s") * NC + lax.axis_index("c")
        base = wid * PER_W
        pltpu.sync_copy(idx_hbm.at[pl.ds(base, PER_W)], idx_v)
        pltpu.sync_copy(tgt_hbm.at[pl.ds(base, PER_W)], tgt_v)

        @pl.loop(0, PER_W, step=LZW)
        def _(j):
            sl = pl.ds(j, LZW)
            pltpu.make_async_copy(
                lz_hbm.at[idx_v.at[sl]], lz_v.at[sl], lzsem).start()

        rows = (rows0, rows1)
        tails = (tail0, tail1)
        gsem = (g0, g1)
        ssem = (s0, s1)

        def _gather_copies(c, b):
            isl = idx_v.at[pl.ds(c * GW, GW)]
            cps = []
            for t in range(7):
                cs = pl.ds(t * 128, 128)
                cps.append(pltpu.make_async_copy(
                    table_hbm.at[:, cs].at[isl], rows[b].at[:, cs], gsem[b]))
            cps.append(pltpu.make_async_copy(
                table_hbm.at[:, pl.ds(TAIL0, 128)].at[isl], tails[b], gsem[b]))
            return cps

        def _write_copies(c, b):
            dst_rows = pl.ds(base + c * GW, GW)
            cps = []
            for t in range(7):
                cs = pl.ds(t * 128, 128)
                cps.append(pltpu.make_async_copy(
                    rows[b].at[:, cs], out_hbm.at[dst_rows, cs], ssem[b]))
            ct = pl.ds(TAIL0, TAILW)
            cps.append(pltpu.make_async_copy(
                rows[b].at[:, ct], out_hbm.at[dst_rows, ct], ssem[b]))
            return cps

        def gather_start(c, b):
            for cp in _gather_copies(c, b):
                cp.start()

        def gather_wait(c, b):
            for cp in _gather_copies(c, b):
                cp.wait()

        def write_start(c, b):
            for cp in _write_copies(c, b):
                cp.start()

        def write_wait(c, b):
            for cp in _write_copies(c, b):
                cp.wait()

        gather_start(0, 0)
        gather_start(1, 1)
        acc_v[...] = jnp.zeros((L,), jnp.float32)

        @pl.loop(0, n_chunks, step=2)
        def _(c0):
            for b in range(2):
                c = c0 + b
                gather_wait(c, b)

                @pl.loop(0, GW)
                def _(r):
                    for kk in (0, 16, 32, 48, 64, 80, TAILW - 16):
                        rows[b][r, pl.ds(TAIL0 + kk, 16)] = (
                            tails[b][r, pl.ds(kk, 16)])

                write_start(c, b)
                for j in range(0, GW, L):
                    rowi = jnp.arange(L, dtype=jnp.int32) + j
                    colt = tgt_v[pl.ds(c * GW + j, L)]
                    vals = plsc.load_gather(rows[b], [rowi, colt])
                    acc_v[...] = acc_v[...] - vals
            for b in range(2):
                nxt = c0 + 2 + b

                @pl.when(nxt < n_chunks)
                def _():
                    write_wait(c0 + b, b)
                    gather_start(nxt, b)

        @pl.loop(0, PER_W, step=LZW)
        def _(j):
            sl = pl.ds(j, LZW)
            pltpu.make_async_copy(
                lz_hbm.at[idx_v.at[sl]], lz_v.at[sl], lzsem).wait()

        @pl.loop(0, PER_W, step=L)
        def _(j):
            acc_v[...] = acc_v[...] + lz_v[pl.ds(j, L)]

        pltpu.sync_copy(acc_v, parts_hbm.at[wid])
        write_wait(n_chunks - 2, 0)
        write_wait(n_chunks - 1, 1)

    return k(table_pad, rowlz, idx_flat, tgt_flat)


def _tr_body(x_ref, o_ref):
    o_ref[...] = x_ref[...].T


def _transpose(h):
    n = h.shape[0]
    blk = 1024
    return pl.pallas_call(
        _tr_body,
        grid=(n // blk,),
        in_specs=[pl.BlockSpec((blk, C), lambda i: (i, 0))],
        out_specs=pl.BlockSpec((C, blk), lambda i: (0, i)),
        out_shape=jax.ShapeDtypeStruct((C, n), jnp.float32),
    )(h)


def _reduce_body(p_ref, o_ref):
    o_ref[...] = (jnp.sum(p_ref[...]) / N).reshape(1, 1)


def _reduce_loss(parts):
    return pl.pallas_call(
        _reduce_body,
        out_shape=jax.ShapeDtypeStruct((1, 1), jnp.float32),
    )(parts)


def kernel(idx, targets, table):
    idx_flat = idx.reshape(-1).astype(jnp.int32)
    tgt_flat = targets.reshape(-1).astype(jnp.int32)
    rowlz, table_pad = _prep(table)
    logits_rm, parts = _gather_and_parts(table_pad, rowlz, idx_flat, tgt_flat)
    logits2 = _transpose(logits_rm).T
    loss = _reduce_loss(parts)
    return (logits2, loss[0, 0])
